# baseline (speedup 1.0000x reference)
import functools

import jax
import jax.numpy as jnp
from jax import lax
from jax.experimental import pallas as pl
from jax.experimental.pallas import tpu as pltpu
from jax.experimental.pallas import tpu_sc as plsc

_D = 768
_E = 64
_N = 32768
_NCHUNK = 2
_NT = _N // _NCHUNK
_BLK = 4096
_NC = 2
_NS = 16
_NW = _NC * _NS
_TPW = _NT // _NW
_L = 16
_G = 4
_CH = 256
_NCH = _TPW // _CH


def _matmul_body(x_ref, wt_ref, b_ref, k_ref):
    x = x_ref[...]
    wt = wt_ref[...]
    s = lax.dot_general(wt, x, (((0,), (1,)), ((), ())),
                        preferred_element_type=jnp.float32)
    s = s + b_ref[...]
    bits = lax.bitcast_convert_type(s, jnp.int32)
    key = jnp.where(bits >= 0, bits, bits ^ 0x7FFFFFFF)
    iota_e = lax.broadcasted_iota(jnp.int32, s.shape, 0)
    k_ref[...] = (key & ~0x3F) | (63 - iota_e)


def _tc_keys(inputs, wt, bcol, chunk):
    nsteps = _NT // _BLK
    return pl.pallas_call(
        _matmul_body,
        grid=(nsteps,),
        in_specs=[
            pl.BlockSpec((_BLK, _D),
                         lambda i, c=chunk: (c * (_NT // _BLK) + i, 0)),
            pl.BlockSpec((_D, _E), lambda i: (0, 0)),
            pl.BlockSpec((_E, 1), lambda i: (0, 0)),
        ],
        out_specs=pl.BlockSpec((_E, _BLK), lambda i: (0, i)),
        out_shape=jax.ShapeDtypeStruct((_E, _NT), jnp.int32),
        compiler_params=pltpu.CompilerParams(
            dimension_semantics=("arbitrary",),
        ),
    )(inputs, wt, bcol)


def _decode(k):
    e = 63 - (k & 0x3F)
    kf = k | 0x20
    bits = jnp.where(kf >= 0, kf, kf ^ 0x7FFFFFFF)
    s = lax.bitcast_convert_type(bits, jnp.float32)
    return s, e


def _sc_body(k_hbm, p_hbm, i_hbm, kbuf, pbuf, ibuf, sem0, sem1):
    wid = lax.axis_index("s") * _NC + lax.axis_index("c")
    base = wid * _TPW
    sems = (sem0, sem1)

    lanes = lax.iota(jnp.int32, _L)
    minint = jnp.full((_L,), -0x80000000, jnp.int32)

    def start(ch, nb):
        off = pl.multiple_of(base + ch * _CH, 128)
        pltpu.async_copy(
            k_hbm.at[:, pl.ds(off, _CH)], kbuf.at[nb], sems[nb])

    def wait(ch, nb):
        off = pl.multiple_of(base + ch * _CH, 128)
        pltpu.make_async_copy(
            k_hbm.at[:, pl.ds(off, _CH)], kbuf.at[nb], sems[nb]).wait()

    def compute(ch, nb):
        kb = kbuf.at[nb]

        def blk_body(blk, carry):
            t0 = blk * (_G * _L)
            sl = [pl.ds(t0 + c * _L, _L) for c in range(_G)]
            m1 = [kb[0, sl[c]] for c in range(_G)]
            m2 = [minint] * _G
            for e in range(1, _E):
                v = [kb[e, sl[c]] for c in range(_G)]
                for c in range(_G):
                    m2[c] = jnp.maximum(m2[c], jnp.minimum(v[c], m1[c]))
                    m1[c] = jnp.maximum(m1[c], v[c])
            for c in range(_G):
                s1, e1 = _decode(m1[c])
                s2, e2 = _decode(m2[c])
                x2 = jnp.exp(s2 - s1)
                p1 = 1.0 / (1.0 + x2)
                p2 = 1.0 - p1
                out = (ch * _CH + t0 + c * _L + lanes) * 2
                plsc.store_scatter(pbuf, [out], p1)
                plsc.store_scatter(pbuf, [out + 1], p2)
                plsc.store_scatter(ibuf, [out], e1)
                plsc.store_scatter(ibuf, [out + 1], e2)
            return carry

        lax.fori_loop(0, _CH // (_G * _L), blk_body, 0)

    start(0, 0)

    def chunk(ch, carry):
        def phase(nb, other):
            @pl.when(ch + 1 < _NCH)
            def _():
                start(ch + 1, other)
            wait(ch, nb)
            compute(ch, nb)
            return 0

        lax.cond(ch % 2 == 0, lambda: phase(0, 1), lambda: phase(1, 0))
        return carry

    lax.fori_loop(0, _NCH, chunk, 0)

    pltpu.sync_copy(pbuf, p_hbm.at[pl.ds(base * 2, _TPW * 2)])
    pltpu.sync_copy(ibuf, i_hbm.at[pl.ds(base * 2, _TPW * 2)])


def _sc_topk(keys):
    mesh = plsc.VectorSubcoreMesh(
        core_axis_name="c", subcore_axis_name="s",
        num_cores=_NC, num_subcores=_NS)
    return pl.kernel(
        _sc_body,
        out_type=[
            jax.ShapeDtypeStruct((_NT * 2,), jnp.float32),
            jax.ShapeDtypeStruct((_NT * 2,), jnp.int32),
        ],
        mesh=mesh,
        compiler_params=pltpu.CompilerParams(needs_layout_passes=False),
        scratch_types=[
            pltpu.VMEM((2, _E, _CH), jnp.int32),
            pltpu.VMEM((_TPW * 2,), jnp.float32),
            pltpu.VMEM((_TPW * 2,), jnp.int32),
            pltpu.SemaphoreType.DMA,
            pltpu.SemaphoreType.DMA,
        ],
    )(keys)


def kernel(inputs, W, b):
    wt = W.T
    bcol = b.reshape(_E, 1)
    pcs, ics = [], []
    for c in range(_NCHUNK):
        keys = _tc_keys(inputs, wt, bcol, c)
        p, i = _sc_topk(keys)
        pcs.append(p)
        ics.append(i)
    probs = jnp.concatenate(pcs).reshape(_N, 2)
    idx = jnp.concatenate(ics).reshape(_N, 2)
    return (probs, idx)


# speedup vs baseline: 1.1954x; 1.1954x over previous
import functools

import jax
import jax.numpy as jnp
from jax import lax
from jax.experimental import pallas as pl
from jax.experimental.pallas import tpu as pltpu
from jax.experimental.pallas import tpu_sc as plsc

_D = 768
_E = 64
_N = 32768
_NCHUNK = 2
_NT = _N // _NCHUNK
_BLK = 4096
_NC = 2
_NS = 16
_NW = _NC * _NS
_TPW = _NT // _NW
_L = 16
_G = 4
_CH = 256
_NCH = _TPW // _CH


def _matmul_body(x_ref, wt_ref, b_ref, k_ref):
    x = x_ref[...]
    wt = wt_ref[...]
    s = lax.dot_general(wt, x, (((0,), (1,)), ((), ())),
                        preferred_element_type=jnp.float32)
    s = s + b_ref[...]
    bits = lax.bitcast_convert_type(s, jnp.int32)
    key = jnp.where(bits >= 0, bits, bits ^ 0x7FFFFFFF)
    iota_e = lax.broadcasted_iota(jnp.int32, s.shape, 0)
    k_ref[...] = (key & ~0x3F) | (63 - iota_e)


def _tc_keys(inputs, wt, bcol, chunk):
    nsteps = _NT // _BLK
    return pl.pallas_call(
        _matmul_body,
        grid=(nsteps,),
        in_specs=[
            pl.BlockSpec((_BLK, _D),
                         lambda i, c=chunk: (c * (_NT // _BLK) + i, 0)),
            pl.BlockSpec((_D, _E), lambda i: (0, 0)),
            pl.BlockSpec((_E, 1), lambda i: (0, 0)),
        ],
        out_specs=pl.BlockSpec((_E, _BLK), lambda i: (0, i)),
        out_shape=jax.ShapeDtypeStruct((_E, _NT), jnp.int32),
        compiler_params=pltpu.CompilerParams(
            dimension_semantics=("arbitrary",),
        ),
    )(inputs, wt, bcol)


def _decode(k):
    e = 63 - (k & 0x3F)
    kf = k | 0x20
    bits = jnp.where(kf >= 0, kf, kf ^ 0x7FFFFFFF)
    s = lax.bitcast_convert_type(bits, jnp.float32)
    return s, e


def _sc_body(k_hbm, p_hbm, i_hbm, kbuf, pbuf, ibuf, sem0, sem1):
    wid = lax.axis_index("s") * _NC + lax.axis_index("c")
    base = wid * _TPW
    sems = (sem0, sem1)

    lanes = lax.iota(jnp.int32, _L)
    minint = jnp.full((_L,), -0x80000000, jnp.int32)

    def start(ch, nb):
        off = pl.multiple_of(base + ch * _CH, 128)
        pltpu.async_copy(
            k_hbm.at[:, pl.ds(off, _CH)], kbuf.at[nb], sems[nb])

    def wait(ch, nb):
        off = pl.multiple_of(base + ch * _CH, 128)
        pltpu.make_async_copy(
            k_hbm.at[:, pl.ds(off, _CH)], kbuf.at[nb], sems[nb]).wait()

    def compute(ch, nb):
        kb = kbuf.at[nb]

        def blk_body(blk, carry):
            t0 = blk * (_G * _L)
            sl = [pl.ds(t0 + c * _L, _L) for c in range(_G)]
            m1 = [kb[0, sl[c]] for c in range(_G)]
            m2 = [minint] * _G
            for e in range(1, _E):
                v = [kb[e, sl[c]] for c in range(_G)]
                for c in range(_G):
                    m2[c] = jnp.maximum(m2[c], jnp.minimum(v[c], m1[c]))
                    m1[c] = jnp.maximum(m1[c], v[c])
            for c in range(_G):
                s1, e1 = _decode(m1[c])
                s2, e2 = _decode(m2[c])
                x2 = jnp.exp(s2 - s1)
                p1 = 1.0 / (1.0 + x2)
                p2 = 1.0 - p1
                out = ch * _CH + t0 + c * _L + lanes
                plsc.store_scatter(pbuf, [out], p1)
                plsc.store_scatter(pbuf, [out + _TPW], p2)
                plsc.store_scatter(ibuf, [out], e1)
                plsc.store_scatter(ibuf, [out + _TPW], e2)
            return carry

        lax.fori_loop(0, _CH // (_G * _L), blk_body, 0)

    start(0, 0)

    def chunk(ch, carry):
        def phase(nb, other):
            @pl.when(ch + 1 < _NCH)
            def _():
                start(ch + 1, other)
            wait(ch, nb)
            compute(ch, nb)
            return 0

        lax.cond(ch % 2 == 0, lambda: phase(0, 1), lambda: phase(1, 0))
        return carry

    lax.fori_loop(0, _NCH, chunk, 0)

    pltpu.sync_copy(pbuf.at[pl.ds(0, _TPW)], p_hbm.at[pl.ds(base, _TPW)])
    pltpu.sync_copy(pbuf.at[pl.ds(_TPW, _TPW)],
                    p_hbm.at[pl.ds(_NT + base, _TPW)])
    pltpu.sync_copy(ibuf.at[pl.ds(0, _TPW)], i_hbm.at[pl.ds(base, _TPW)])
    pltpu.sync_copy(ibuf.at[pl.ds(_TPW, _TPW)],
                    i_hbm.at[pl.ds(_NT + base, _TPW)])


def _sc_topk(keys):
    mesh = plsc.VectorSubcoreMesh(
        core_axis_name="c", subcore_axis_name="s",
        num_cores=_NC, num_subcores=_NS)
    return pl.kernel(
        _sc_body,
        out_type=[
            jax.ShapeDtypeStruct((_NT * 2,), jnp.float32),
            jax.ShapeDtypeStruct((_NT * 2,), jnp.int32),
        ],
        mesh=mesh,
        compiler_params=pltpu.CompilerParams(needs_layout_passes=False),
        scratch_types=[
            pltpu.VMEM((2, _E, _CH), jnp.int32),
            pltpu.VMEM((_TPW * 2,), jnp.float32),
            pltpu.VMEM((_TPW * 2,), jnp.int32),
            pltpu.SemaphoreType.DMA,
            pltpu.SemaphoreType.DMA,
        ],
    )(keys)


_EB = 4096


def _exp_body(p_ref, i_ref, po_ref, io_ref):
    po_ref[...] = p_ref[...].T
    io_ref[...] = i_ref[...].T


def _expand(p2, i2):
    return pl.pallas_call(
        _exp_body,
        grid=(_N // _EB,),
        in_specs=[
            pl.BlockSpec((2, _EB), lambda i: (0, i)),
            pl.BlockSpec((2, _EB), lambda i: (0, i)),
        ],
        out_specs=[
            pl.BlockSpec((_EB, 2), lambda i: (i, 0)),
            pl.BlockSpec((_EB, 2), lambda i: (i, 0)),
        ],
        out_shape=[
            jax.ShapeDtypeStruct((_N, 2), jnp.float32),
            jax.ShapeDtypeStruct((_N, 2), jnp.int32),
        ],
        compiler_params=pltpu.CompilerParams(
            dimension_semantics=("arbitrary",),
        ),
    )(p2, i2)


def kernel(inputs, W, b):
    wt = W.T
    bcol = b.reshape(_E, 1)
    pcs, ics = [], []
    for c in range(_NCHUNK):
        keys = _tc_keys(inputs, wt, bcol, c)
        p, i = _sc_topk(keys)
        pcs.append(p.reshape(2, _NT))
        ics.append(i.reshape(2, _NT))
    p2 = jnp.concatenate(pcs, axis=1)
    i2 = jnp.concatenate(ics, axis=1)
    return _expand(p2, i2)
